# baseline (speedup 1.0000x reference)
import functools

import jax
import jax.numpy as jnp
from jax import lax
from jax.experimental import pallas as pl
from jax.experimental.pallas import tpu as pltpu
from jax.experimental.pallas import tpu_sc as plsc

H = 256
AF = 133
BF = 147
NM = 250
L = 200
NA = 1 + NM * L
NB = 100001
DEG = 6
DEPTH = 5

NW = 32
WA = 16
WB = 56
NA_P = 51200
NB_P = 100352
ZA = NA
ZB = NB
CA = NA_P // (NW * WA)
CB = NB_P // (NW * WB)
GA = WA * DEG

_f32 = jnp.float32


def _proj_pad(x, w, n_pad, bn):
    n, k = x.shape
    m = w.shape[1]
    assert n_pad % bn == 0 and (n_pad // bn - 1) * bn < n

    def body(x_ref, w_ref, o_ref):
        acc = jnp.dot(x_ref[...], w_ref[...], preferred_element_type=_f32)
        row = (pl.program_id(0) * bn
               + jax.lax.broadcasted_iota(jnp.int32, (bn, 1), 0))
        o_ref[...] = jnp.where(row < n, jnp.maximum(acc, 0.0), 0.0)

    return pl.pallas_call(
        body,
        grid=(n_pad // bn,),
        in_specs=[pl.BlockSpec((bn, k), lambda i: (i, 0)),
                  pl.BlockSpec((k, m), lambda i: (0, 0))],
        out_specs=pl.BlockSpec((bn, m), lambda i: (i, 0)),
        out_shape=jax.ShapeDtypeStruct((n_pad, m), _f32),
    )(x, w)


def _mm_bias_relu(x, w, add=None, bias=None, relu=True, bn=2048):
    n, k = x.shape
    m = w.shape[1]
    assert n % bn == 0

    def body(*refs):
        if add is not None:
            x_ref, w_ref, a_ref = refs[:3]
            o_ref = refs[3]
        else:
            x_ref, w_ref = refs[:2]
            a_ref = None
            o_ref = refs[2]
        acc = jnp.dot(x_ref[...], w_ref[...], preferred_element_type=_f32)
        if a_ref is not None:
            acc = acc + a_ref[...]
        if bias is not None:
            acc = acc + bias
        if relu:
            acc = jnp.maximum(acc, 0.0)
        o_ref[...] = acc

    in_specs = [
        pl.BlockSpec((bn, k), lambda i: (i, 0)),
        pl.BlockSpec((k, m), lambda i: (0, 0)),
    ]
    args = [x, w]
    if add is not None:
        in_specs.append(pl.BlockSpec((bn, m), lambda i: (i, 0)))
        args.append(add)
    return pl.pallas_call(
        body,
        grid=(n // bn,),
        in_specs=in_specs,
        out_specs=pl.BlockSpec((bn, m), lambda i: (i, 0)),
        out_shape=jax.ShapeDtypeStruct((n, m), _f32),
    )(*args)


def _wlr_stage(agg, ma, ia, w1, w2, w3, gbias, bn=1024):
    n = agg.shape[0]

    def body(a_ref, m_ref, i_ref, w1_ref, w2_ref, w3_ref, gb_ref,
             hid_ref, msg_ref):
        acc = jnp.dot(a_ref[...], w1_ref[...], preferred_element_type=_f32)
        acc += jnp.dot(m_ref[...], w2_ref[...], preferred_element_type=_f32)
        acc += jnp.dot(i_ref[...], w3_ref[...], preferred_element_type=_f32)
        hid_ref[...] = acc
        msg_ref[...] = jnp.maximum(acc + gb_ref[...], 0.0)

    bs_x = pl.BlockSpec((bn, H), lambda i: (i, 0))
    bs_w = pl.BlockSpec((H, H), lambda i: (0, 0))
    bs_b = pl.BlockSpec((1, H), lambda i: (0, 0))
    return pl.pallas_call(
        body,
        grid=(n // bn,),
        in_specs=[bs_x, bs_x, bs_x, bs_w, bs_w, bs_w, bs_b],
        out_specs=[bs_x, bs_x],
        out_shape=[jax.ShapeDtypeStruct((n, H), _f32),
                   jax.ShapeDtypeStruct((n, H), _f32)],
    )(agg, ma, ia, w1, w2, w3, gbias)


def _h0_max(hidden):
    def body(x_ref, o_ref):
        o_ref[0, 0] = jnp.max(x_ref[...], axis=0)

    out = pl.pallas_call(
        body,
        grid=(NM,),
        in_specs=[pl.BlockSpec((L, H), lambda i: (i, 0))],
        out_specs=pl.BlockSpec((1, 1, H), lambda i: (i, 0, 0)),
        out_shape=jax.ShapeDtypeStruct((NM, 1, H), _f32),
    )(hidden)
    return out.reshape(NM, H)


def _gi_proj(message, wf_t, wr_t, bf, br, bn=2000):
    n = NM * L

    def body(x_ref, wf_ref, wr_ref, bf_ref, br_ref, gf_ref, gr_ref):
        x = x_ref[...]
        gf_ref[...] = (jnp.dot(x, wf_ref[...], preferred_element_type=_f32)
                       + bf_ref[...])
        gr_ref[...] = (jnp.dot(x, wr_ref[...], preferred_element_type=_f32)
                       + br_ref[...])

    bs_x = pl.BlockSpec((bn, H), lambda i: (i, 0))
    bs_w = pl.BlockSpec((H, 3 * H), lambda i: (0, 0))
    bs_b = pl.BlockSpec((1, 3 * H), lambda i: (0, 0))
    bs_g = pl.BlockSpec((bn, 3 * H), lambda i: (i, 0))
    return pl.pallas_call(
        body,
        grid=(n // bn,),
        in_specs=[bs_x, bs_w, bs_w, bs_b, bs_b],
        out_specs=[bs_g, bs_g],
        out_shape=[jax.ShapeDtypeStruct((n, 3 * H), _f32),
                   jax.ShapeDtypeStruct((n, 3 * H), _f32)],
    )(message, wf_t, wr_t, bf, br)


def _gru_step(gi, gh, h):
    r = jax.nn.sigmoid(gi[:, 0:H] + gh[:, 0:H])
    z = jax.nn.sigmoid(gi[:, H:2 * H] + gh[:, H:2 * H])
    n = jnp.tanh(gi[:, 2 * H:] + r * gh[:, 2 * H:])
    return (1.0 - z) * n + z * h


TB = 8


def _bigru(gi_f, gi_r, h0, whhf_t, whhr_t, bhf, bhr):
    gf3 = gi_f.reshape(NM, L, 3 * H)
    gb3 = gi_r.reshape(NM, L, 3 * H)
    nblk = L // TB

    def body(gf_ref, gb_ref, h0_ref, wf_ref, wr_ref, bf_ref, br_ref,
             of_ref, ob_ref, hf_ref, hb_ref):
        i = pl.program_id(0)

        @pl.when(i == 0)
        def _():
            hf_ref[...] = h0_ref[...]
            hb_ref[...] = h0_ref[...]

        hf = hf_ref[...]
        hb = hb_ref[...]
        wf = wf_ref[...]
        wr = wr_ref[...]
        for tt in range(TB):
            ghf = jnp.dot(hf, wf, preferred_element_type=_f32) + bf_ref[...]
            hf = _gru_step(gf_ref[:, tt, :], ghf, hf)
            of_ref[:, tt, :] = hf.astype(jnp.bfloat16)

            bt = TB - 1 - tt
            ghb = jnp.dot(hb, wr, preferred_element_type=_f32) + br_ref[...]
            hb = _gru_step(gb_ref[:, bt, :], ghb, hb)
            ob_ref[:, bt, :] = hb.astype(jnp.bfloat16)
        hf_ref[...] = hf
        hb_ref[...] = hb

    bs_gf = pl.BlockSpec((NM, TB, 3 * H), lambda i: (0, i, 0))
    bs_gb = pl.BlockSpec((NM, TB, 3 * H), lambda i: (0, nblk - 1 - i, 0))
    bs_h0 = pl.BlockSpec((NM, H), lambda i: (0, 0))
    bs_w = pl.BlockSpec((H, 3 * H), lambda i: (0, 0))
    bs_b = pl.BlockSpec((1, 3 * H), lambda i: (0, 0))
    bs_of = pl.BlockSpec((NM, TB, H), lambda i: (0, i, 0))
    bs_ob = pl.BlockSpec((NM, TB, H), lambda i: (0, nblk - 1 - i, 0))
    out_f, out_b = pl.pallas_call(
        body,
        grid=(nblk,),
        in_specs=[bs_gf, bs_gb, bs_h0, bs_w, bs_w, bs_b, bs_b],
        out_specs=[bs_of, bs_ob],
        out_shape=[jax.ShapeDtypeStruct((NM, L, H), jnp.bfloat16),
                   jax.ShapeDtypeStruct((NM, L, H), jnp.bfloat16)],
        scratch_shapes=[pltpu.VMEM((NM, H), _f32), pltpu.VMEM((NM, H), _f32)],
    )(gf3, gb3, h0, whhf_t, whhr_t, bhf, bhr)
    return out_f.reshape(NM * L, H), out_b.reshape(NM * L, H)


def _out_proj(out_f, out_b, m0, wo1, wo2, wo_sum, b_o, bn=2000):
    n = NM * L

    bft = jnp.bfloat16

    def body(f_ref, b_ref, m0_ref, w1_ref, w2_ref, ws_ref, bo_ref,
             o_ref, h_ref):
        t = pl.program_id(0)
        acc = jnp.dot(f_ref[...], w1_ref[...].astype(bft),
                      preferred_element_type=_f32)
        acc += jnp.dot(b_ref[...], w2_ref[...].astype(bft),
                       preferred_element_type=_f32)
        o_ref[...] = jnp.maximum(acc + bo_ref[...], 0.0)

        @pl.when(t == 0)
        def _():
            h = jnp.dot(m0_ref[...], ws_ref[...], preferred_element_type=_f32)
            h_ref[...] = jnp.maximum(h + bo_ref[...], 0.0)

    bs_x = pl.BlockSpec((bn, H), lambda t: (t, 0))
    bs_m0 = pl.BlockSpec((1, H), lambda t: (0, 0))
    bs_w = pl.BlockSpec((H, H), lambda t: (0, 0))
    return pl.pallas_call(
        body,
        grid=(n // bn,),
        in_specs=[bs_x, bs_x, bs_m0, bs_w, bs_w, bs_w, bs_m0],
        out_specs=[bs_x, bs_m0],
        out_shape=[jax.ShapeDtypeStruct((n, H), _f32),
                   jax.ShapeDtypeStruct((1, H), _f32)],
    )(out_f, out_b, m0, wo1, wo2, wo_sum, b_o)


def _sc_mesh():
    return plsc.VectorSubcoreMesh(core_axis_name="c", subcore_axis_name="s")


def _sc_agg(mb, ma, idx_flat, rev_idx, final):
    scratch = [
        pltpu.VMEM((CA * GA,), jnp.int32),
        pltpu.VMEM((GA, H), _f32),
        pltpu.VMEM((GA, H), _f32),
        pltpu.VMEM((WA, H), _f32),
        pltpu.VMEM((WA, H), _f32),
        pltpu.SemaphoreType.DMA,
        pltpu.SemaphoreType.DMA,
        pltpu.SemaphoreType.DMA,
        pltpu.SemaphoreType.DMA,
    ]
    if final:
        out_type = jax.ShapeDtypeStruct((NA_P, H), _f32)
    else:
        out_type = [jax.ShapeDtypeStruct((NA_P, H), _f32),
                    jax.ShapeDtypeStruct((NB_P, H), _f32)]
        scratch += [
            pltpu.VMEM((WA, H), _f32),
            pltpu.VMEM((WA, H), _f32),
            pltpu.SemaphoreType.DMA,
            pltpu.SemaphoreType.DMA,
            pltpu.VMEM((CB * WB,), jnp.int32),
            pltpu.VMEM((WB, H), _f32),
            pltpu.VMEM((WB, H), _f32),
            pltpu.SemaphoreType.DMA,
            pltpu.SemaphoreType.DMA,
            pltpu.SemaphoreType.DMA,
            pltpu.SemaphoreType.DMA,
        ]

    @functools.partial(
        pl.kernel,
        mesh=_sc_mesh(),
        out_type=out_type,
        scratch_types=scratch,
    )
    def k(*refs):
        if final:
            (mb_hbm, ma_hbm, idx_hbm, rix_hbm, out_hbm,
             idx_all, rows0, rows1, out0, out1, sr0, sr1, so0, so1) = refs
        else:
            (mb_hbm, ma_hbm, idx_hbm, rix_hbm, out_hbm, rev_hbm,
             idx_all, rows0, rows1, out0, out1, sr0, sr1, so0, so1,
             mav0, mav1, sm0, sm1,
             rix_all, rv0, rv1, srg0, srg1, srs0, srs1) = refs
            mav = (mav0, mav1)
            sm = (sm0, sm1)
            rv = (rv0, rv1)
            srg = (srg0, srg1)
            srs = (srs0, srs1)
        rows = (rows0, rows1)
        outb = (out0, out1)
        sr = (sr0, sr1)
        so = (so0, so1)
        wid = lax.axis_index("s") * 2 + lax.axis_index("c")
        base0 = wid * CA
        pltpu.sync_copy(idx_hbm.at[pl.ds(base0 * GA, CA * GA)], idx_all)
        if not final:
            rbase0 = wid * CB
            pltpu.sync_copy(
                rix_hbm.at[pl.ds(rbase0 * WB, CB * WB)], rix_all)

        def issue(j, p):
            pltpu.async_copy(
                mb_hbm.at[idx_all.at[pl.ds(j * GA, GA)]], rows[p], sr[p])
            if not final:
                pltpu.async_copy(
                    ma_hbm.at[pl.ds((base0 + j) * WA, WA)], mav[p], sm[p])

        def rev_issue(j, p):
            pltpu.async_copy(
                mb_hbm.at[rix_all.at[pl.ds(j * WB, WB)]], rv[p], srg[p])

        issue(0, 0)
        if not final:
            rev_issue(0, 0)

        @pl.loop(0, CA, step=2)
        def _(j0):
            for b in range(2):
                j = j0 + b
                p = b
                q = 1 - b

                @pl.when(j + 1 < CA)
                def _():
                    issue(j + 1, q)

                if not final:
                    @pl.when(j < CB)
                    def _():
                        pltpu.make_async_copy(
                            mb_hbm.at[pl.ds(0, WB)], rv[p], srg[p]).wait()
                        pltpu.async_copy(
                            rv[p],
                            rev_hbm.at[pl.ds((rbase0 + j) * WB, WB)], srs[p])

                    @pl.when(j + 1 < CB)
                    def _():
                        @pl.when(j >= 1)
                        def _():
                            pltpu.make_async_copy(
                                rv[q], rev_hbm.at[pl.ds(0, WB)], srs[q]
                            ).wait()
                        rev_issue(j + 1, q)

                pltpu.make_async_copy(
                    mb_hbm.at[idx_all.at[pl.ds(0, GA)]], rows[p], sr[p]
                ).wait()
                if not final:
                    pltpu.make_async_copy(
                        ma_hbm.at[pl.ds(0, WA)], mav[p], sm[p]).wait()

                @pl.when(j >= 2)
                def _():
                    pltpu.make_async_copy(
                        outb[p], out_hbm.at[pl.ds(0, WA)], so[p]).wait()

                @pl.loop(0, WA)
                def _(a):
                    r = a * DEG

                    @pl.loop(0, H, step=32)
                    def _(c0):
                        for u in range(2):
                            c = c0 + u * 16
                            v0 = rows[p][r, pl.ds(c, 16)]
                            s = v0
                            m = v0
                            for kk in range(1, DEG):
                                v = rows[p][r + kk, pl.ds(c, 16)]
                                s = s + v
                                m = jnp.maximum(m, v)
                            g = s * m
                            if final:
                                outb[p][a, pl.ds(c, 16)] = g
                            else:
                                outb[p][a, pl.ds(c, 16)] = (
                                    mav[p][a, pl.ds(c, 16)] + g)

                pltpu.async_copy(
                    outb[p], out_hbm.at[pl.ds((base0 + j) * WA, WA)], so[p])

        for p in range(2):
            pltpu.make_async_copy(
                outb[p], out_hbm.at[pl.ds(0, WA)], so[p]).wait()
        if not final:
            for p in range(2):
                pltpu.make_async_copy(
                    rv[p], rev_hbm.at[pl.ds(0, WB)], srs[p]).wait()

    return k(mb, ma, idx_flat, rev_idx)


def _sc_pre(ma, rev, b2a_r):
    @functools.partial(
        pl.kernel,
        mesh=_sc_mesh(),
        out_type=jax.ShapeDtypeStruct((NB_P, H), _f32),
        scratch_types=[
            pltpu.VMEM((CB * WB,), jnp.int32),
            pltpu.VMEM((WB, H), _f32),
            pltpu.VMEM((WB, H), _f32),
            pltpu.VMEM((WB, H), _f32),
            pltpu.VMEM((WB, H), _f32),
            pltpu.VMEM((WB, H), _f32),
            pltpu.VMEM((WB, H), _f32),
            pltpu.SemaphoreType.DMA,
            pltpu.SemaphoreType.DMA,
            pltpu.SemaphoreType.DMA,
            pltpu.SemaphoreType.DMA,
        ],
    )
    def k(ma_hbm, rev_hbm, ixa_hbm, pre_hbm,
          ixa_all, ga0, ga1, gb0, gb1, po0, po1,
          sg0, sg1, so0, so1):
        ga = (ga0, ga1)
        gb = (gb0, gb1)
        po = (po0, po1)
        sg = (sg0, sg1)
        so = (so0, so1)
        wid = lax.axis_index("s") * 2 + lax.axis_index("c")
        base0 = wid * CB
        pltpu.sync_copy(ixa_hbm.at[pl.ds(base0 * WB, CB * WB)], ixa_all)

        def issue(j, p):
            pltpu.async_copy(
                ma_hbm.at[ixa_all.at[pl.ds(j * WB, WB)]], ga[p], sg[p])
            pltpu.async_copy(
                rev_hbm.at[pl.ds((base0 + j) * WB, WB)], gb[p], sg[p])

        issue(0, 0)

        @pl.loop(0, CB, step=2)
        def _(j0):
            for b in range(2):
                j = j0 + b
                p = b
                q = 1 - b

                @pl.when(j + 1 < CB)
                def _():
                    issue(j + 1, q)

                pltpu.make_async_copy(
                    ma_hbm.at[pl.ds(0, WB)], ga[p], sg[p]).wait()
                pltpu.make_async_copy(
                    rev_hbm.at[pl.ds(0, WB)], gb[p], sg[p]).wait()

                @pl.when(j >= 2)
                def _():
                    pltpu.make_async_copy(
                        po[p], pre_hbm.at[pl.ds(0, WB)], so[p]).wait()

                @pl.loop(0, WB)
                def _(a):
                    @pl.loop(0, H, step=32)
                    def _(c0):
                        for u in range(2):
                            c = c0 + u * 16
                            po[p][a, pl.ds(c, 16)] = (
                                ga[p][a, pl.ds(c, 16)]
                                - gb[p][a, pl.ds(c, 16)])

                pltpu.async_copy(
                    po[p], pre_hbm.at[pl.ds((base0 + j) * WB, WB)], so[p])

        for p in range(2):
            pltpu.make_async_copy(
                po[p], pre_hbm.at[pl.ds(0, WB)], so[p]).wait()

    return k(ma, rev, b2a_r)


def kernel(f_atoms, f_bonds, a2b, b2a, b2revb, a_scope, W_i_atom, W_i_bond,
           W_h, W_lr, gru_bias, w_ih_f, w_hh_f, b_ih_f, b_hh_f, w_ih_r,
           w_hh_r, b_ih_r, b_hh_r, W_o, b_o):
    a2b = a2b.astype(jnp.int32)
    b2a = b2a.astype(jnp.int32)
    b2revb = b2revb.astype(jnp.int32)

    a2b_r = jnp.where(a2b == 0, ZB, a2b)
    a2b_r = jnp.roll(a2b_r, -1, axis=0)
    a2b_r = jnp.pad(a2b_r, ((0, NA_P - NA), (0, 0)), constant_values=ZB)
    a2b_flat = a2b_r.reshape(-1)
    b2a_n = jnp.where(b2a == 0, NM * L, b2a - 1)
    b2a_r = jnp.pad(b2a_n, (0, NB_P - NB), constant_values=ZA)
    b2revb_r = jnp.pad(b2revb, (0, NB_P - NB), constant_values=ZB)

    fa = jnp.roll(f_atoms, -1, axis=0)
    ia = _proj_pad(fa, W_i_atom, NA_P, bn=1280)
    ib = _proj_pad(f_bonds, W_i_bond, NB_P, bn=2048)

    ma = ia
    mb = ib
    for d in range(DEPTH - 1):
        ma, rev = _sc_agg(mb, ma, a2b_flat, b2revb_r, final=False)
        pre = _sc_pre(ma, rev, b2a_r)
        mb = _mm_bias_relu(pre, W_h[d], add=ib, bn=2048)
    agg = _sc_agg(mb, ma, a2b_flat, b2revb_r, final=True)

    hidden, message = _wlr_stage(
        agg, ma, ia, W_lr[0:H], W_lr[H:2 * H], W_lr[2 * H:],
        gru_bias[None, :], bn=1280)

    h0 = _h0_max(hidden)
    gi_f, gi_r = _gi_proj(message, w_ih_f.T, w_ih_r.T,
                          b_ih_f[None, :], b_ih_r[None, :])
    out_f, out_b = _bigru(gi_f, gi_r, h0, w_hh_f.T, w_hh_r.T,
                          b_hh_f[None, :], b_hh_r[None, :])

    body, head = _out_proj(out_f, out_b, message[NM * L:NM * L + 1],
                           W_o[0:H], W_o[H:], W_o[0:H] + W_o[H:],
                           b_o[None, :])
    return jnp.concatenate([head, body], axis=0)


# speedup vs baseline: 1.2303x; 1.2303x over previous
import functools

import jax
import jax.numpy as jnp
from jax import lax
from jax.experimental import pallas as pl
from jax.experimental.pallas import tpu as pltpu
from jax.experimental.pallas import tpu_sc as plsc

H = 256
AF = 133
BF = 147
NM = 250
L = 200
NA = 1 + NM * L
NB = 100001
DEG = 6
DEPTH = 5

NW = 32
WA = 24
WB = 56
NA_P = 50688
NB_P = 100352
ZA = NA
ZB = NB
CA = NA_P // (NW * WA)
CB = NB_P // (NW * WB)
GA = WA * DEG

_f32 = jnp.float32


def _proj_pad(x, w, n_pad, bn):
    n, k = x.shape
    m = w.shape[1]
    assert n_pad % bn == 0 and (n_pad // bn - 1) * bn < n

    def body(x_ref, w_ref, o_ref):
        acc = jnp.dot(x_ref[...], w_ref[...], preferred_element_type=_f32)
        row = (pl.program_id(0) * bn
               + jax.lax.broadcasted_iota(jnp.int32, (bn, 1), 0))
        o_ref[...] = jnp.where(row < n, jnp.maximum(acc, 0.0), 0.0)

    return pl.pallas_call(
        body,
        grid=(n_pad // bn,),
        in_specs=[pl.BlockSpec((bn, k), lambda i: (i, 0)),
                  pl.BlockSpec((k, m), lambda i: (0, 0))],
        out_specs=pl.BlockSpec((bn, m), lambda i: (i, 0)),
        out_shape=jax.ShapeDtypeStruct((n_pad, m), _f32),
    )(x, w)


def _mm_bias_relu(x, w, add=None, bias=None, relu=True, bn=2048):
    n, k = x.shape
    m = w.shape[1]
    assert n % bn == 0

    def body(*refs):
        if add is not None:
            x_ref, w_ref, a_ref = refs[:3]
            o_ref = refs[3]
        else:
            x_ref, w_ref = refs[:2]
            a_ref = None
            o_ref = refs[2]
        acc = jnp.dot(x_ref[...], w_ref[...], preferred_element_type=_f32)
        if a_ref is not None:
            acc = acc + a_ref[...]
        if bias is not None:
            acc = acc + bias
        if relu:
            acc = jnp.maximum(acc, 0.0)
        o_ref[...] = acc

    in_specs = [
        pl.BlockSpec((bn, k), lambda i: (i, 0)),
        pl.BlockSpec((k, m), lambda i: (0, 0)),
    ]
    args = [x, w]
    if add is not None:
        in_specs.append(pl.BlockSpec((bn, m), lambda i: (i, 0)))
        args.append(add)
    return pl.pallas_call(
        body,
        grid=(n // bn,),
        in_specs=in_specs,
        out_specs=pl.BlockSpec((bn, m), lambda i: (i, 0)),
        out_shape=jax.ShapeDtypeStruct((n, m), _f32),
    )(*args)


def _wlr_stage(agg, ma, ia, w1, w2, w3, gbias, bn=1024):
    n = agg.shape[0]

    def body(a_ref, m_ref, i_ref, w1_ref, w2_ref, w3_ref, gb_ref,
             hid_ref, msg_ref):
        acc = jnp.dot(a_ref[...], w1_ref[...], preferred_element_type=_f32)
        acc += jnp.dot(m_ref[...], w2_ref[...], preferred_element_type=_f32)
        acc += jnp.dot(i_ref[...], w3_ref[...], preferred_element_type=_f32)
        hid_ref[...] = acc
        msg_ref[...] = jnp.maximum(acc + gb_ref[...], 0.0)

    bs_x = pl.BlockSpec((bn, H), lambda i: (i, 0))
    bs_w = pl.BlockSpec((H, H), lambda i: (0, 0))
    bs_b = pl.BlockSpec((1, H), lambda i: (0, 0))
    return pl.pallas_call(
        body,
        grid=(n // bn,),
        in_specs=[bs_x, bs_x, bs_x, bs_w, bs_w, bs_w, bs_b],
        out_specs=[bs_x, bs_x],
        out_shape=[jax.ShapeDtypeStruct((n, H), _f32),
                   jax.ShapeDtypeStruct((n, H), _f32)],
    )(agg, ma, ia, w1, w2, w3, gbias)


def _h0_max(hidden):
    def body(x_ref, o_ref):
        o_ref[0, 0] = jnp.max(x_ref[...], axis=0)

    out = pl.pallas_call(
        body,
        grid=(NM,),
        in_specs=[pl.BlockSpec((L, H), lambda i: (i, 0))],
        out_specs=pl.BlockSpec((1, 1, H), lambda i: (i, 0, 0)),
        out_shape=jax.ShapeDtypeStruct((NM, 1, H), _f32),
    )(hidden)
    return out.reshape(NM, H)


def _gi_proj(message, wf_t, wr_t, bf, br, bn=2000):
    n = NM * L

    def body(x_ref, wf_ref, wr_ref, bf_ref, br_ref, gf_ref, gr_ref):
        x = x_ref[...]
        gf_ref[...] = (jnp.dot(x, wf_ref[...], preferred_element_type=_f32)
                       + bf_ref[...])
        gr_ref[...] = (jnp.dot(x, wr_ref[...], preferred_element_type=_f32)
                       + br_ref[...])

    bs_x = pl.BlockSpec((bn, H), lambda i: (i, 0))
    bs_w = pl.BlockSpec((H, 3 * H), lambda i: (0, 0))
    bs_b = pl.BlockSpec((1, 3 * H), lambda i: (0, 0))
    bs_g = pl.BlockSpec((bn, 3 * H), lambda i: (i, 0))
    return pl.pallas_call(
        body,
        grid=(n // bn,),
        in_specs=[bs_x, bs_w, bs_w, bs_b, bs_b],
        out_specs=[bs_g, bs_g],
        out_shape=[jax.ShapeDtypeStruct((n, 3 * H), _f32),
                   jax.ShapeDtypeStruct((n, 3 * H), _f32)],
    )(message, wf_t, wr_t, bf, br)


def _gru_step(gi, gh, h):
    r = jax.nn.sigmoid(gi[:, 0:H] + gh[:, 0:H])
    z = jax.nn.sigmoid(gi[:, H:2 * H] + gh[:, H:2 * H])
    n = jnp.tanh(gi[:, 2 * H:] + r * gh[:, 2 * H:])
    return (1.0 - z) * n + z * h


TB = 8


def _bigru(gi_f, gi_r, h0, whhf_t, whhr_t, bhf, bhr):
    gf3 = gi_f.reshape(NM, L, 3 * H)
    gb3 = gi_r.reshape(NM, L, 3 * H)
    nblk = L // TB

    def body(gf_ref, gb_ref, h0_ref, wf_ref, wr_ref, bf_ref, br_ref,
             of_ref, ob_ref, hf_ref, hb_ref):
        i = pl.program_id(0)

        @pl.when(i == 0)
        def _():
            hf_ref[...] = h0_ref[...]
            hb_ref[...] = h0_ref[...]

        hf = hf_ref[...]
        hb = hb_ref[...]
        wf = wf_ref[...]
        wr = wr_ref[...]
        for tt in range(TB):
            ghf = jnp.dot(hf, wf, preferred_element_type=_f32) + bf_ref[...]
            hf = _gru_step(gf_ref[:, tt, :], ghf, hf)
            of_ref[:, tt, :] = hf.astype(jnp.bfloat16)

            bt = TB - 1 - tt
            ghb = jnp.dot(hb, wr, preferred_element_type=_f32) + br_ref[...]
            hb = _gru_step(gb_ref[:, bt, :], ghb, hb)
            ob_ref[:, bt, :] = hb.astype(jnp.bfloat16)
        hf_ref[...] = hf
        hb_ref[...] = hb

    bs_gf = pl.BlockSpec((NM, TB, 3 * H), lambda i: (0, i, 0))
    bs_gb = pl.BlockSpec((NM, TB, 3 * H), lambda i: (0, nblk - 1 - i, 0))
    bs_h0 = pl.BlockSpec((NM, H), lambda i: (0, 0))
    bs_w = pl.BlockSpec((H, 3 * H), lambda i: (0, 0))
    bs_b = pl.BlockSpec((1, 3 * H), lambda i: (0, 0))
    bs_of = pl.BlockSpec((NM, TB, H), lambda i: (0, i, 0))
    bs_ob = pl.BlockSpec((NM, TB, H), lambda i: (0, nblk - 1 - i, 0))
    out_f, out_b = pl.pallas_call(
        body,
        grid=(nblk,),
        in_specs=[bs_gf, bs_gb, bs_h0, bs_w, bs_w, bs_b, bs_b],
        out_specs=[bs_of, bs_ob],
        out_shape=[jax.ShapeDtypeStruct((NM, L, H), jnp.bfloat16),
                   jax.ShapeDtypeStruct((NM, L, H), jnp.bfloat16)],
        scratch_shapes=[pltpu.VMEM((NM, H), _f32), pltpu.VMEM((NM, H), _f32)],
    )(gf3, gb3, h0, whhf_t, whhr_t, bhf, bhr)
    return out_f.reshape(NM * L, H), out_b.reshape(NM * L, H)


def _out_proj(out_f, out_b, m0, wo1, wo2, wo_sum, b_o, bn=2000):
    n = NM * L

    bft = jnp.bfloat16

    def body(f_ref, b_ref, m0_ref, w1_ref, w2_ref, ws_ref, bo_ref,
             o_ref, h_ref):
        t = pl.program_id(0)
        acc = jnp.dot(f_ref[...], w1_ref[...].astype(bft),
                      preferred_element_type=_f32)
        acc += jnp.dot(b_ref[...], w2_ref[...].astype(bft),
                       preferred_element_type=_f32)
        o_ref[...] = jnp.maximum(acc + bo_ref[...], 0.0)

        @pl.when(t == 0)
        def _():
            h = jnp.dot(m0_ref[...], ws_ref[...], preferred_element_type=_f32)
            h_ref[...] = jnp.maximum(h + bo_ref[...], 0.0)

    bs_x = pl.BlockSpec((bn, H), lambda t: (t, 0))
    bs_m0 = pl.BlockSpec((1, H), lambda t: (0, 0))
    bs_w = pl.BlockSpec((H, H), lambda t: (0, 0))
    return pl.pallas_call(
        body,
        grid=(n // bn,),
        in_specs=[bs_x, bs_x, bs_m0, bs_w, bs_w, bs_w, bs_m0],
        out_specs=[bs_x, bs_m0],
        out_shape=[jax.ShapeDtypeStruct((n, H), _f32),
                   jax.ShapeDtypeStruct((1, H), _f32)],
    )(out_f, out_b, m0, wo1, wo2, wo_sum, b_o)


def _sc_mesh():
    return plsc.VectorSubcoreMesh(core_axis_name="c", subcore_axis_name="s")


def _sc_agg(mb, ma, idx_flat, final):
    GH = GA // 2

    scratch = [
        pltpu.VMEM((CA * GA,), jnp.int32),
        pltpu.VMEM((GA, H), _f32),
        pltpu.VMEM((GA, H), _f32),
        pltpu.VMEM((WA, H), _f32),
        pltpu.VMEM((WA, H), _f32),
        pltpu.SemaphoreType.DMA,
        pltpu.SemaphoreType.DMA,
        pltpu.SemaphoreType.DMA,
        pltpu.SemaphoreType.DMA,
    ]
    if not final:
        scratch += [
            pltpu.VMEM((WA, H), _f32),
            pltpu.VMEM((WA, H), _f32),
            pltpu.SemaphoreType.DMA,
            pltpu.SemaphoreType.DMA,
        ]

    @functools.partial(
        pl.kernel,
        mesh=_sc_mesh(),
        out_type=jax.ShapeDtypeStruct((NA_P, H), _f32),
        scratch_types=scratch,
    )
    def k(mb_hbm, ma_hbm, idx_hbm, out_hbm, idx_all, rows0, rows1,
          out0, out1, sr0, sr1, so0, so1, *rest):
        if not final:
            mav0, mav1, sm0, sm1 = rest
            mav = (mav0, mav1)
            sm = (sm0, sm1)
        rows = (rows0, rows1)
        outb = (out0, out1)
        sr = (sr0, sr1)
        so = (so0, so1)
        wid = lax.axis_index("s") * 2 + lax.axis_index("c")
        base0 = wid * CA
        pltpu.sync_copy(idx_hbm.at[pl.ds(base0 * GA, CA * GA)], idx_all)

        def issue(j, p):
            pltpu.async_copy(
                mb_hbm.at[idx_all.at[pl.ds(j * GA, GH)]],
                rows[p].at[pl.ds(0, GH)], sr[p])
            pltpu.async_copy(
                mb_hbm.at[idx_all.at[pl.ds(j * GA + GH, GH)]],
                rows[p].at[pl.ds(GH, GH)], sr[p])
            if not final:
                pltpu.async_copy(
                    ma_hbm.at[pl.ds((base0 + j) * WA, WA)], mav[p], sm[p])

        issue(0, 0)

        @pl.loop(0, CA, step=2)
        def _(j0):
            for b in range(2):
                j = j0 + b
                p = b
                q = 1 - b

                @pl.when(j + 1 < CA)
                def _():
                    issue(j + 1, q)

                pltpu.make_async_copy(
                    mb_hbm.at[idx_all.at[pl.ds(0, GA)]], rows[p], sr[p]
                ).wait()
                if not final:
                    pltpu.make_async_copy(
                        ma_hbm.at[pl.ds(0, WA)], mav[p], sm[p]).wait()

                @pl.when(j >= 2)
                def _():
                    pltpu.make_async_copy(
                        outb[p], out_hbm.at[pl.ds(0, WA)], so[p]).wait()

                @pl.loop(0, WA)
                def _(a):
                    r = a * DEG

                    @pl.loop(0, H, step=32)
                    def _(c0):
                        for u in range(2):
                            c = c0 + u * 16
                            v0 = rows[p][r, pl.ds(c, 16)]
                            s = v0
                            m = v0
                            for kk in range(1, DEG):
                                v = rows[p][r + kk, pl.ds(c, 16)]
                                s = s + v
                                m = jnp.maximum(m, v)
                            g = s * m
                            if final:
                                outb[p][a, pl.ds(c, 16)] = g
                            else:
                                outb[p][a, pl.ds(c, 16)] = (
                                    mav[p][a, pl.ds(c, 16)] + g)

                pltpu.async_copy(
                    outb[p], out_hbm.at[pl.ds((base0 + j) * WA, WA)], so[p])

        for p in range(2):
            pltpu.make_async_copy(
                outb[p], out_hbm.at[pl.ds(0, WA)], so[p]).wait()

    return k(mb, ma, idx_flat)


def _sc_pre(ma, mb, b2a_r, b2revb_r):
    @functools.partial(
        pl.kernel,
        mesh=_sc_mesh(),
        out_type=jax.ShapeDtypeStruct((NB_P, H), _f32),
        scratch_types=[
            pltpu.VMEM((CB * WB,), jnp.int32),
            pltpu.VMEM((CB * WB,), jnp.int32),
            pltpu.VMEM((WB, H), _f32),
            pltpu.VMEM((WB, H), _f32),
            pltpu.VMEM((WB, H), _f32),
            pltpu.VMEM((WB, H), _f32),
            pltpu.VMEM((WB, H), _f32),
            pltpu.VMEM((WB, H), _f32),
            pltpu.SemaphoreType.DMA,
            pltpu.SemaphoreType.DMA,
            pltpu.SemaphoreType.DMA,
            pltpu.SemaphoreType.DMA,
        ],
    )
    def k(ma_hbm, mb_hbm, ixa_hbm, ixb_hbm, pre_hbm,
          ixa_all, ixb_all, ga0, ga1, gb0, gb1, po0, po1,
          sg0, sg1, so0, so1):
        ga = (ga0, ga1)
        gb = (gb0, gb1)
        po = (po0, po1)
        sg = (sg0, sg1)
        so = (so0, so1)
        wid = lax.axis_index("s") * 2 + lax.axis_index("c")
        base0 = wid * CB
        pltpu.sync_copy(ixa_hbm.at[pl.ds(base0 * WB, CB * WB)], ixa_all)
        pltpu.sync_copy(ixb_hbm.at[pl.ds(base0 * WB, CB * WB)], ixb_all)

        def issue(j, p):
            pltpu.async_copy(
                ma_hbm.at[ixa_all.at[pl.ds(j * WB, WB)]], ga[p], sg[p])
            pltpu.async_copy(
                mb_hbm.at[ixb_all.at[pl.ds(j * WB, WB)]], gb[p], sg[p])

        issue(0, 0)

        @pl.loop(0, CB, step=2)
        def _(j0):
            for b in range(2):
                j = j0 + b
                p = b
                q = 1 - b

                @pl.when(j + 1 < CB)
                def _():
                    issue(j + 1, q)

                pltpu.make_async_copy(
                    ma_hbm.at[pl.ds(0, WB)], ga[p], sg[p]).wait()
                pltpu.make_async_copy(
                    mb_hbm.at[pl.ds(0, WB)], gb[p], sg[p]).wait()

                @pl.when(j >= 2)
                def _():
                    pltpu.make_async_copy(
                        po[p], pre_hbm.at[pl.ds(0, WB)], so[p]).wait()

                @pl.loop(0, WB)
                def _(a):
                    @pl.loop(0, H, step=32)
                    def _(c0):
                        for u in range(2):
                            c = c0 + u * 16
                            po[p][a, pl.ds(c, 16)] = (
                                ga[p][a, pl.ds(c, 16)]
                                - gb[p][a, pl.ds(c, 16)])

                pltpu.async_copy(
                    po[p], pre_hbm.at[pl.ds((base0 + j) * WB, WB)], so[p])

        for p in range(2):
            pltpu.make_async_copy(
                po[p], pre_hbm.at[pl.ds(0, WB)], so[p]).wait()

    return k(ma, mb, b2a_r, b2revb_r)


def kernel(f_atoms, f_bonds, a2b, b2a, b2revb, a_scope, W_i_atom, W_i_bond,
           W_h, W_lr, gru_bias, w_ih_f, w_hh_f, b_ih_f, b_hh_f, w_ih_r,
           w_hh_r, b_ih_r, b_hh_r, W_o, b_o):
    a2b = a2b.astype(jnp.int32)
    b2a = b2a.astype(jnp.int32)
    b2revb = b2revb.astype(jnp.int32)

    a2b_r = jnp.where(a2b == 0, ZB, a2b)
    a2b_r = jnp.roll(a2b_r, -1, axis=0)
    a2b_r = jnp.pad(a2b_r, ((0, NA_P - NA), (0, 0)), constant_values=ZB)
    a2b_flat = a2b_r.reshape(-1)
    b2a_n = jnp.where(b2a == 0, NM * L, b2a - 1)
    b2a_r = jnp.pad(b2a_n, (0, NB_P - NB), constant_values=ZA)
    b2revb_r = jnp.pad(b2revb, (0, NB_P - NB), constant_values=ZB)

    fa = jnp.roll(f_atoms, -1, axis=0)
    ia = _proj_pad(fa, W_i_atom, NA_P, bn=768)
    ib = _proj_pad(f_bonds, W_i_bond, NB_P, bn=2048)

    ma = ia
    mb = ib
    for d in range(DEPTH - 1):
        ma = _sc_agg(mb, ma, a2b_flat, final=False)
        pre = _sc_pre(ma, mb, b2a_r, b2revb_r)
        mb = _mm_bias_relu(pre, W_h[d], add=ib, bn=2048)
    agg = _sc_agg(mb, ma, a2b_flat, final=True)

    hidden, message = _wlr_stage(
        agg, ma, ia, W_lr[0:H], W_lr[H:2 * H], W_lr[2 * H:],
        gru_bias[None, :], bn=768)

    h0 = _h0_max(hidden)
    gi_f, gi_r = _gi_proj(message, w_ih_f.T, w_ih_r.T,
                          b_ih_f[None, :], b_ih_r[None, :])
    out_f, out_b = _bigru(gi_f, gi_r, h0, w_hh_f.T, w_hh_r.T,
                          b_hh_f[None, :], b_hh_r[None, :])

    body, head = _out_proj(out_f, out_b, message[NM * L:NM * L + 1],
                           W_o[0:H], W_o[H:], W_o[0:H] + W_o[H:],
                           b_o[None, :])
    return jnp.concatenate([head, body], axis=0)


# speedup vs baseline: 1.2331x; 1.0023x over previous
import functools

import jax
import jax.numpy as jnp
from jax import lax
from jax.experimental import pallas as pl
from jax.experimental.pallas import tpu as pltpu
from jax.experimental.pallas import tpu_sc as plsc

H = 256
AF = 133
BF = 147
NM = 250
L = 200
NA = 1 + NM * L
NB = 100001
DEG = 6
DEPTH = 5

NW = 32
WA = 24
WB = 56
NA_P = 50688
NB_P = 100352
ZA = NA
ZB = NB
CA = NA_P // (NW * WA)
CB = NB_P // (NW * WB)
GA = WA * DEG

_f32 = jnp.float32


def _proj_pad(x, w, n_pad, bn):
    n, k = x.shape
    m = w.shape[1]
    assert n_pad % bn == 0 and (n_pad // bn - 1) * bn < n

    def body(x_ref, w_ref, o_ref):
        acc = jnp.dot(x_ref[...], w_ref[...], preferred_element_type=_f32)
        row = (pl.program_id(0) * bn
               + jax.lax.broadcasted_iota(jnp.int32, (bn, 1), 0))
        o_ref[...] = jnp.where(row < n, jnp.maximum(acc, 0.0), 0.0)

    return pl.pallas_call(
        body,
        grid=(n_pad // bn,),
        in_specs=[pl.BlockSpec((bn, k), lambda i: (i, 0)),
                  pl.BlockSpec((k, m), lambda i: (0, 0))],
        out_specs=pl.BlockSpec((bn, m), lambda i: (i, 0)),
        out_shape=jax.ShapeDtypeStruct((n_pad, m), _f32),
    )(x, w)


def _mm_bias_relu(x, w, add=None, bias=None, relu=True, bn=2048):
    n, k = x.shape
    m = w.shape[1]
    assert n % bn == 0

    def body(*refs):
        if add is not None:
            x_ref, w_ref, a_ref = refs[:3]
            o_ref = refs[3]
        else:
            x_ref, w_ref = refs[:2]
            a_ref = None
            o_ref = refs[2]
        acc = jnp.dot(x_ref[...], w_ref[...], preferred_element_type=_f32)
        if a_ref is not None:
            acc = acc + a_ref[...]
        if bias is not None:
            acc = acc + bias
        if relu:
            acc = jnp.maximum(acc, 0.0)
        o_ref[...] = acc

    in_specs = [
        pl.BlockSpec((bn, k), lambda i: (i, 0)),
        pl.BlockSpec((k, m), lambda i: (0, 0)),
    ]
    args = [x, w]
    if add is not None:
        in_specs.append(pl.BlockSpec((bn, m), lambda i: (i, 0)))
        args.append(add)
    return pl.pallas_call(
        body,
        grid=(n // bn,),
        in_specs=in_specs,
        out_specs=pl.BlockSpec((bn, m), lambda i: (i, 0)),
        out_shape=jax.ShapeDtypeStruct((n, m), _f32),
    )(*args)


def _wlr_stage(agg, ma, ia, w1, w2, w3, gbias, bn=1024):
    n = agg.shape[0]

    def body(a_ref, m_ref, i_ref, w1_ref, w2_ref, w3_ref, gb_ref,
             hid_ref, msg_ref):
        acc = jnp.dot(a_ref[...], w1_ref[...], preferred_element_type=_f32)
        acc += jnp.dot(m_ref[...], w2_ref[...], preferred_element_type=_f32)
        acc += jnp.dot(i_ref[...], w3_ref[...], preferred_element_type=_f32)
        hid_ref[...] = acc
        msg_ref[...] = jnp.maximum(acc + gb_ref[...], 0.0)

    bs_x = pl.BlockSpec((bn, H), lambda i: (i, 0))
    bs_w = pl.BlockSpec((H, H), lambda i: (0, 0))
    bs_b = pl.BlockSpec((1, H), lambda i: (0, 0))
    return pl.pallas_call(
        body,
        grid=(n // bn,),
        in_specs=[bs_x, bs_x, bs_x, bs_w, bs_w, bs_w, bs_b],
        out_specs=[bs_x, bs_x],
        out_shape=[jax.ShapeDtypeStruct((n, H), _f32),
                   jax.ShapeDtypeStruct((n, H), _f32)],
    )(agg, ma, ia, w1, w2, w3, gbias)


def _h0_max(hidden):
    def body(x_ref, o_ref):
        o_ref[0, 0] = jnp.max(x_ref[...], axis=0)

    out = pl.pallas_call(
        body,
        grid=(NM,),
        in_specs=[pl.BlockSpec((L, H), lambda i: (i, 0))],
        out_specs=pl.BlockSpec((1, 1, H), lambda i: (i, 0, 0)),
        out_shape=jax.ShapeDtypeStruct((NM, 1, H), _f32),
    )(hidden)
    return out.reshape(NM, H)


def _gi_proj(message, wf_t, wr_t, bf, br, bn=2000):
    n = NM * L

    def body(x_ref, wf_ref, wr_ref, bf_ref, br_ref, gf_ref, gr_ref):
        x = x_ref[...]
        gf_ref[...] = (jnp.dot(x, wf_ref[...], preferred_element_type=_f32)
                       + bf_ref[...])
        gr_ref[...] = (jnp.dot(x, wr_ref[...], preferred_element_type=_f32)
                       + br_ref[...])

    bs_x = pl.BlockSpec((bn, H), lambda i: (i, 0))
    bs_w = pl.BlockSpec((H, 3 * H), lambda i: (0, 0))
    bs_b = pl.BlockSpec((1, 3 * H), lambda i: (0, 0))
    bs_g = pl.BlockSpec((bn, 3 * H), lambda i: (i, 0))
    return pl.pallas_call(
        body,
        grid=(n // bn,),
        in_specs=[bs_x, bs_w, bs_w, bs_b, bs_b],
        out_specs=[bs_g, bs_g],
        out_shape=[jax.ShapeDtypeStruct((n, 3 * H), _f32),
                   jax.ShapeDtypeStruct((n, 3 * H), _f32)],
    )(message, wf_t, wr_t, bf, br)


def _gru_step(gi, gh, h):
    r = jax.nn.sigmoid(gi[:, 0:H] + gh[:, 0:H])
    z = jax.nn.sigmoid(gi[:, H:2 * H] + gh[:, H:2 * H])
    n = jnp.tanh(gi[:, 2 * H:] + r * gh[:, 2 * H:])
    return (1.0 - z) * n + z * h


TB = 8


def _bigru(gi_f, gi_r, h0, whhf_t, whhr_t, bhf, bhr):
    gf3 = gi_f.reshape(NM, L, 3 * H)
    gb3 = gi_r.reshape(NM, L, 3 * H)
    nblk = L // TB

    def body(gf_ref, gb_ref, h0_ref, wf_ref, wr_ref, bf_ref, br_ref,
             of_ref, ob_ref, hf_ref, hb_ref):
        i = pl.program_id(0)

        @pl.when(i == 0)
        def _():
            hf_ref[...] = h0_ref[...]
            hb_ref[...] = h0_ref[...]

        hf = hf_ref[...]
        hb = hb_ref[...]
        wf = wf_ref[...]
        wr = wr_ref[...]
        for tt in range(TB):
            ghf = jnp.dot(hf, wf, preferred_element_type=_f32) + bf_ref[...]
            hf = _gru_step(gf_ref[:, tt, :], ghf, hf)
            of_ref[:, tt, :] = hf.astype(jnp.bfloat16)

            bt = TB - 1 - tt
            ghb = jnp.dot(hb, wr, preferred_element_type=_f32) + br_ref[...]
            hb = _gru_step(gb_ref[:, bt, :], ghb, hb)
            ob_ref[:, bt, :] = hb.astype(jnp.bfloat16)
        hf_ref[...] = hf
        hb_ref[...] = hb

    bs_gf = pl.BlockSpec((NM, TB, 3 * H), lambda i: (0, i, 0))
    bs_gb = pl.BlockSpec((NM, TB, 3 * H), lambda i: (0, nblk - 1 - i, 0))
    bs_h0 = pl.BlockSpec((NM, H), lambda i: (0, 0))
    bs_w = pl.BlockSpec((H, 3 * H), lambda i: (0, 0))
    bs_b = pl.BlockSpec((1, 3 * H), lambda i: (0, 0))
    bs_of = pl.BlockSpec((NM, TB, H), lambda i: (0, i, 0))
    bs_ob = pl.BlockSpec((NM, TB, H), lambda i: (0, nblk - 1 - i, 0))
    out_f, out_b = pl.pallas_call(
        body,
        grid=(nblk,),
        in_specs=[bs_gf, bs_gb, bs_h0, bs_w, bs_w, bs_b, bs_b],
        out_specs=[bs_of, bs_ob],
        out_shape=[jax.ShapeDtypeStruct((NM, L, H), jnp.bfloat16),
                   jax.ShapeDtypeStruct((NM, L, H), jnp.bfloat16)],
        scratch_shapes=[pltpu.VMEM((NM, H), _f32), pltpu.VMEM((NM, H), _f32)],
    )(gf3, gb3, h0, whhf_t, whhr_t, bhf, bhr)
    return out_f.reshape(NM * L, H), out_b.reshape(NM * L, H)


def _out_proj(out_f, out_b, m0, wo1, wo2, wo_sum, b_o, bn=2000):
    n = NM * L

    bft = jnp.bfloat16

    def body(f_ref, b_ref, m0_ref, w1_ref, w2_ref, ws_ref, bo_ref,
             o_ref, h_ref):
        t = pl.program_id(0)
        acc = jnp.dot(f_ref[...], w1_ref[...].astype(bft),
                      preferred_element_type=_f32)
        acc += jnp.dot(b_ref[...], w2_ref[...].astype(bft),
                       preferred_element_type=_f32)
        o_ref[...] = jnp.maximum(acc + bo_ref[...], 0.0)

        @pl.when(t == 0)
        def _():
            h = jnp.dot(m0_ref[...], ws_ref[...], preferred_element_type=_f32)
            h_ref[...] = jnp.maximum(h + bo_ref[...], 0.0)

    bs_x = pl.BlockSpec((bn, H), lambda t: (t, 0))
    bs_m0 = pl.BlockSpec((1, H), lambda t: (0, 0))
    bs_w = pl.BlockSpec((H, H), lambda t: (0, 0))
    return pl.pallas_call(
        body,
        grid=(n // bn,),
        in_specs=[bs_x, bs_x, bs_m0, bs_w, bs_w, bs_w, bs_m0],
        out_specs=[bs_x, bs_m0],
        out_shape=[jax.ShapeDtypeStruct((n, H), _f32),
                   jax.ShapeDtypeStruct((1, H), _f32)],
    )(out_f, out_b, m0, wo1, wo2, wo_sum, b_o)


def _sc_mesh():
    return plsc.VectorSubcoreMesh(core_axis_name="c", subcore_axis_name="s")


def _sc_agg(mb, ma, idx_flat, final):
    GH = GA // 2

    scratch = [
        pltpu.VMEM((CA * GA,), jnp.int32),
        pltpu.VMEM((GA, H), _f32),
        pltpu.VMEM((GA, H), _f32),
        pltpu.VMEM((WA, H), _f32),
        pltpu.VMEM((WA, H), _f32),
        pltpu.SemaphoreType.DMA,
        pltpu.SemaphoreType.DMA,
        pltpu.SemaphoreType.DMA,
        pltpu.SemaphoreType.DMA,
    ]
    if not final:
        scratch += [
            pltpu.VMEM((WA, H), _f32),
            pltpu.VMEM((WA, H), _f32),
            pltpu.SemaphoreType.DMA,
            pltpu.SemaphoreType.DMA,
        ]

    @functools.partial(
        pl.kernel,
        mesh=_sc_mesh(),
        out_type=jax.ShapeDtypeStruct((NA_P, H), _f32),
        scratch_types=scratch,
    )
    def k(mb_hbm, ma_hbm, idx_hbm, out_hbm, idx_all, rows0, rows1,
          out0, out1, sr0, sr1, so0, so1, *rest):
        if not final:
            mav0, mav1, sm0, sm1 = rest
            mav = (mav0, mav1)
            sm = (sm0, sm1)
        rows = (rows0, rows1)
        outb = (out0, out1)
        sr = (sr0, sr1)
        so = (so0, so1)
        wid = lax.axis_index("s") * 2 + lax.axis_index("c")
        base0 = wid * CA
        pltpu.sync_copy(idx_hbm.at[pl.ds(base0 * GA, CA * GA)], idx_all)

        def issue(j, p):
            pltpu.async_copy(
                mb_hbm.at[idx_all.at[pl.ds(j * GA, GH)]],
                rows[p].at[pl.ds(0, GH)], sr[p])
            pltpu.async_copy(
                mb_hbm.at[idx_all.at[pl.ds(j * GA + GH, GH)]],
                rows[p].at[pl.ds(GH, GH)], sr[p])
            if not final:
                pltpu.async_copy(
                    ma_hbm.at[pl.ds((base0 + j) * WA, WA)], mav[p], sm[p])

        issue(0, 0)

        @pl.loop(0, CA, step=2)
        def _(j0):
            for b in range(2):
                j = j0 + b
                p = b
                q = 1 - b

                @pl.when(j + 1 < CA)
                def _():
                    issue(j + 1, q)

                pltpu.make_async_copy(
                    mb_hbm.at[idx_all.at[pl.ds(0, GA)]], rows[p], sr[p]
                ).wait()
                if not final:
                    pltpu.make_async_copy(
                        ma_hbm.at[pl.ds(0, WA)], mav[p], sm[p]).wait()

                @pl.when(j >= 2)
                def _():
                    pltpu.make_async_copy(
                        outb[p], out_hbm.at[pl.ds(0, WA)], so[p]).wait()

                @pl.loop(0, WA)
                def _(a):
                    r = a * DEG

                    @pl.loop(0, H, step=32)
                    def _(c0):
                        for u in range(2):
                            c = c0 + u * 16
                            v0 = rows[p][r, pl.ds(c, 16)]
                            s = v0
                            m = v0
                            for kk in range(1, DEG):
                                v = rows[p][r + kk, pl.ds(c, 16)]
                                s = s + v
                                m = jnp.maximum(m, v)
                            g = s * m
                            if final:
                                outb[p][a, pl.ds(c, 16)] = g
                            else:
                                outb[p][a, pl.ds(c, 16)] = (
                                    mav[p][a, pl.ds(c, 16)] + g)

                pltpu.async_copy(
                    outb[p], out_hbm.at[pl.ds((base0 + j) * WA, WA)], so[p])

        for p in range(2):
            pltpu.make_async_copy(
                outb[p], out_hbm.at[pl.ds(0, WA)], so[p]).wait()

    return k(mb, ma, idx_flat)


def _sc_pre(ma, mb, b2a_r, b2revb_r):
    @functools.partial(
        pl.kernel,
        mesh=_sc_mesh(),
        out_type=jax.ShapeDtypeStruct((NB_P, H), _f32),
        scratch_types=[
            pltpu.VMEM((CB * WB,), jnp.int32),
            pltpu.VMEM((CB * WB,), jnp.int32),
            pltpu.VMEM((WB, H), _f32),
            pltpu.VMEM((WB, H), _f32),
            pltpu.VMEM((WB, H), _f32),
            pltpu.VMEM((WB, H), _f32),
            pltpu.VMEM((WB, H), _f32),
            pltpu.VMEM((WB, H), _f32),
            pltpu.SemaphoreType.DMA,
            pltpu.SemaphoreType.DMA,
            pltpu.SemaphoreType.DMA,
            pltpu.SemaphoreType.DMA,
        ],
    )
    def k(ma_hbm, mb_hbm, ixa_hbm, ixb_hbm, pre_hbm,
          ixa_all, ixb_all, ga0, ga1, gb0, gb1, po0, po1,
          sg0, sg1, so0, so1):
        ga = (ga0, ga1)
        gb = (gb0, gb1)
        po = (po0, po1)
        sg = (sg0, sg1)
        so = (so0, so1)
        wid = lax.axis_index("s") * 2 + lax.axis_index("c")
        base0 = wid * CB
        pltpu.sync_copy(ixa_hbm.at[pl.ds(base0 * WB, CB * WB)], ixa_all)
        pltpu.sync_copy(ixb_hbm.at[pl.ds(base0 * WB, CB * WB)], ixb_all)

        def issue(j, p):
            pltpu.async_copy(
                ma_hbm.at[ixa_all.at[pl.ds(j * WB, WB)]], ga[p], sg[p])
            pltpu.async_copy(
                mb_hbm.at[ixb_all.at[pl.ds(j * WB, WB)]], gb[p], sg[p])

        issue(0, 0)

        @pl.loop(0, CB, step=2)
        def _(j0):
            for b in range(2):
                j = j0 + b
                p = b
                q = 1 - b

                @pl.when(j + 1 < CB)
                def _():
                    issue(j + 1, q)

                pltpu.make_async_copy(
                    ma_hbm.at[pl.ds(0, WB)], ga[p], sg[p]).wait()
                pltpu.make_async_copy(
                    mb_hbm.at[pl.ds(0, WB)], gb[p], sg[p]).wait()

                @pl.when(j >= 2)
                def _():
                    pltpu.make_async_copy(
                        po[p], pre_hbm.at[pl.ds(0, WB)], so[p]).wait()

                @pl.loop(0, WB)
                def _(a):
                    @pl.loop(0, H, step=32)
                    def _(c0):
                        for u in range(2):
                            c = c0 + u * 16
                            po[p][a, pl.ds(c, 16)] = (
                                ga[p][a, pl.ds(c, 16)]
                                - gb[p][a, pl.ds(c, 16)])

                pltpu.async_copy(
                    po[p], pre_hbm.at[pl.ds((base0 + j) * WB, WB)], so[p])

        for p in range(2):
            pltpu.make_async_copy(
                po[p], pre_hbm.at[pl.ds(0, WB)], so[p]).wait()

    return k(ma, mb, b2a_r, b2revb_r)


def kernel(f_atoms, f_bonds, a2b, b2a, b2revb, a_scope, W_i_atom, W_i_bond,
           W_h, W_lr, gru_bias, w_ih_f, w_hh_f, b_ih_f, b_hh_f, w_ih_r,
           w_hh_r, b_ih_r, b_hh_r, W_o, b_o):
    a2b = a2b.astype(jnp.int32)
    b2a = b2a.astype(jnp.int32)
    b2revb = b2revb.astype(jnp.int32)

    a2b_r = jnp.where(a2b == 0, ZB, a2b)
    a2b_r = jnp.roll(a2b_r, -1, axis=0)
    a2b_r = jnp.pad(a2b_r, ((0, NA_P - NA), (0, 0)), constant_values=ZB)
    a2b_flat = a2b_r.reshape(-1)
    b2a_n = jnp.where(b2a == 0, NM * L, b2a - 1)
    b2a_r = jnp.pad(b2a_n, (0, NB_P - NB), constant_values=ZA)
    b2revb_r = jnp.pad(b2revb, (0, NB_P - NB), constant_values=ZB)

    fa = jnp.roll(f_atoms, -1, axis=0)
    ia = _proj_pad(fa, W_i_atom, NA_P, bn=768)
    ib = _proj_pad(f_bonds, W_i_bond, NB_P, bn=3584)

    ma = ia
    mb = ib
    for d in range(DEPTH - 1):
        ma = _sc_agg(mb, ma, a2b_flat, final=False)
        pre = _sc_pre(ma, mb, b2a_r, b2revb_r)
        mb = _mm_bias_relu(pre, W_h[d], add=ib, bn=3584)
    agg = _sc_agg(mb, ma, a2b_flat, final=True)

    hidden, message = _wlr_stage(
        agg, ma, ia, W_lr[0:H], W_lr[H:2 * H], W_lr[2 * H:],
        gru_bias[None, :], bn=768)

    h0 = _h0_max(hidden)
    gi_f, gi_r = _gi_proj(message, w_ih_f.T, w_ih_r.T,
                          b_ih_f[None, :], b_ih_r[None, :])
    out_f, out_b = _bigru(gi_f, gi_r, h0, w_hh_f.T, w_hh_r.T,
                          b_hh_f[None, :], b_hh_r[None, :])

    body, head = _out_proj(out_f, out_b, message[NM * L:NM * L + 1],
                           W_o[0:H], W_o[H:], W_o[0:H] + W_o[H:],
                           b_o[None, :])
    return jnp.concatenate([head, body], axis=0)
